# baseline (speedup 1.0000x reference)
import functools

import jax
import jax.numpy as jnp
from jax import lax
from jax.experimental import pallas as pl
from jax.experimental.pallas import tpu as pltpu
from jax.experimental.pallas import tpu_sc as plsc

NC, NS = 2, 16
NW = NC * NS
D = 128
LANES = 16
NBUF = 3


def _emb_body(T, S, L, tok_hbm, tidx_hbm, sidx_hbm, pos_hbm, seg_hbm,
              out_hbm, comb_sh, idx_v, rows_v, isems, gsems, asems, osems):
    cid = lax.axis_index("c")
    sid = lax.axis_index("s")
    wid = sid * NC + cid
    iota = lax.iota(jnp.int32, LANES)

    n_blocks = L // 8
    pltpu.sync_copy(seg_hbm, rows_v.at[0, pl.ds(8, 2)])

    def build_block(k):
        l0 = pl.multiple_of(k * 8, 8)
        pltpu.sync_copy(pos_hbm.at[pl.ds(l0, 8)], rows_v.at[0, pl.ds(0, 8)])
        for s in range(2):
            def r_body(r, carry):
                for d in range(D // LANES):
                    sl = pl.ds(d * LANES, LANES)
                    rows_v[1, r, sl] = rows_v[0, r, sl] + rows_v[0, 8 + s, sl]
                return carry

            lax.fori_loop(0, 8, r_body, 0)
            pltpu.sync_copy(rows_v.at[1, pl.ds(0, 8)],
                            comb_sh.at[pl.ds(pl.multiple_of(s * L + l0, 8),
                                             8)])

    build_block(sid)

    @pl.when(sid < n_blocks - NS)
    def _():
        build_block(sid + NS)

    plsc.subcore_barrier()

    def idx_start(g, b):
        base = (wid * S + g) * T
        pltpu.make_async_copy(tidx_hbm.at[pl.ds(base, T)], idx_v.at[b, 0],
                              isems.at[b]).start()
        pltpu.make_async_copy(sidx_hbm.at[pl.ds(base, T)], idx_v.at[b, 1],
                              isems.at[b]).start()

    def idx_wait(g, b):
        base = (wid * S + g) * T
        pltpu.make_async_copy(tidx_hbm.at[pl.ds(base, T)], idx_v.at[b, 0],
                              isems.at[b]).wait()
        pltpu.make_async_copy(sidx_hbm.at[pl.ds(base, T)], idx_v.at[b, 1],
                              isems.at[b]).wait()

    def gather_start(b):
        pltpu.make_async_copy(tok_hbm.at[idx_v.at[b, 0]], rows_v.at[b],
                              gsems.at[b]).start()

    def gather_wait(b):
        pltpu.make_async_copy(tok_hbm.at[idx_v.at[b, 0]], rows_v.at[b],
                              gsems.at[b]).wait()

    def cidx_fix(g, b):
        base = (wid * S + g) * T
        for j in range(T // LANES):
            sl = pl.ds(j * LANES, LANES)
            lv = (base + j * LANES + iota) % L
            idx_v[b, 1, sl] = idx_v[b, 1, sl] * L + lv

    def addg(b):
        pltpu.async_copy(comb_sh.at[idx_v.at[b, 1]], rows_v.at[b],
                         asems.at[b], add=True).wait()

    def out_start(g, b):
        base = (wid * S + g) * T
        pltpu.make_async_copy(rows_v.at[b], out_hbm.at[pl.ds(base, T)],
                              osems.at[b]).start()

    def out_wait(g, b):
        base = (wid * S + g) * T
        pltpu.make_async_copy(rows_v.at[b], out_hbm.at[pl.ds(base, T)],
                              osems.at[b]).wait()

    def chunk_iter(g, b, bn, *, first=False, do_next=True, do_idx=True):
        if do_next:
            idx_wait(g + 1, bn)
            if not first:
                out_wait(g - 2, bn)
            gather_start(bn)
        cidx_fix(g, b)
        gather_wait(b)
        addg(b)
        out_start(g, b)
        if do_idx:
            idx_start(g + 3, b)

    idx_start(0, 0)
    idx_wait(0, 0)
    gather_start(0)
    idx_start(1, 1)
    idx_start(2, 2)
    chunk_iter(0, 0, 1, first=True)
    chunk_iter(1, 1, 2, first=True)

    def mid_body(i, carry):
        g = 3 * i + 2
        chunk_iter(g, 2, 0)
        chunk_iter(g + 1, 0, 1)
        chunk_iter(g + 2, 1, 2)
        return carry

    lax.fori_loop(0, (S - 5) // 3, mid_body, 0)

    chunk_iter(S - 3, 2, 0, do_idx=False)
    chunk_iter(S - 2, 0, 1, do_idx=False)
    chunk_iter(S - 1, 1, 2, do_next=False, do_idx=False)
    out_wait(S - 3, 2)
    out_wait(S - 2, 0)
    out_wait(S - 1, 1)


def kernel(input_ids, segment_ids, tok_table, pos_table, seg_table):
    B, L = input_ids.shape

    tidx = input_ids.reshape(-1).astype(jnp.int32)
    sidx = segment_ids.reshape(-1).astype(jnp.int32)

    T = 128
    n_chunks = B * L // T
    S = n_chunks // NW
    assert (S - 5) % 3 == 0

    n_seg = seg_table.shape[0]
    mesh = plsc.VectorSubcoreMesh(core_axis_name="c", subcore_axis_name="s",
                                  num_cores=NC, num_subcores=NS)
    emb = pl.kernel(
        functools.partial(_emb_body, T, S, L),
        out_type=jax.ShapeDtypeStruct((B * L, D), jnp.float32),
        mesh=mesh,
        scratch_types=[
            pltpu.VMEM_SHARED((n_seg * L, D), jnp.float32),
            pltpu.VMEM((NBUF, 2, T), jnp.int32),
            pltpu.VMEM((NBUF, T, D), jnp.float32),
            pltpu.SemaphoreType.DMA((NBUF,)),
            pltpu.SemaphoreType.DMA((NBUF,)),
            pltpu.SemaphoreType.DMA((NBUF,)),
            pltpu.SemaphoreType.DMA((NBUF,)),
        ],
    )
    out = emb(tok_table, tidx, sidx, pos_table, seg_table)
    return out.reshape(B, L, D)
